# Initial kernel scaffold; baseline (speedup 1.0000x reference)
import jax
import jax.numpy as jnp
from jax.experimental import pallas as pl


def kernel(boxes, scores):
    raise NotImplementedError("write your pallas kernel here")


# speedup vs baseline: 36.9204x; 36.9204x over previous
import functools

import jax
import jax.numpy as jnp
from jax import lax
from jax.experimental import pallas as pl
from jax.experimental.pallas import tpu as pltpu

_SCORE_THR = 0.05
_MAX_OUT = 300
_LANES = 128


def _nms_kernel(b_ref, s_ref, out_ref, sw_ref, *, rows):
    x1 = b_ref[0]
    y1 = b_ref[1]
    x2 = b_ref[2]
    y2 = b_ref[3]
    area = (x2 - x1) * (y2 - y1)
    s = s_ref[...]
    sw_ref[...] = jnp.where(s > _SCORE_THR, s, 0.0)
    out_ref[...] = jnp.zeros_like(out_ref)
    lin = (lax.broadcasted_iota(jnp.int32, (rows, _LANES), 0) * _LANES
           + lax.broadcasted_iota(jnp.int32, (rows, _LANES), 1))
    lane = lax.broadcasted_iota(jnp.int32, (1, _LANES), 1)

    def body(i, carry):
        sw = sw_ref[...]
        vmax = jnp.max(sw)

        @pl.when(vmax > _SCORE_THR)
        def _():
            idx = jnp.min(jnp.where(sw == vmax, lin, jnp.int32(2**30)))
            sel = lin == idx
            bx1 = jnp.sum(jnp.where(sel, x1, 0.0))
            by1 = jnp.sum(jnp.where(sel, y1, 0.0))
            bx2 = jnp.sum(jnp.where(sel, x2, 0.0))
            by2 = jnp.sum(jnp.where(sel, y2, 0.0))
            barea = (bx2 - bx1) * (by2 - by1)
            iw = jnp.maximum(jnp.minimum(bx2, x2) - jnp.maximum(bx1, x1), 0.0)
            ih = jnp.maximum(jnp.minimum(by2, y2) - jnp.maximum(by1, y1), 0.0)
            inter = iw * ih
            keep = 3.0 * inter < barea + area + 1e-6
            sw_ref[...] = jnp.where(keep, sw, 0.0)
            det = (jnp.where(lane == 0, bx1, 0.0)
                   + jnp.where(lane == 1, by1, 0.0)
                   + jnp.where(lane == 2, bx2, 0.0)
                   + jnp.where(lane == 3, by2, 0.0)
                   + jnp.where(lane == 4, vmax, 0.0))
            out_ref[pl.ds(i, 1), :] = det

        return carry

    lax.fori_loop(0, _MAX_OUT, body, 0)


def kernel(boxes, scores):
    n = boxes.shape[0]
    rows = -(-n // _LANES)
    rows = -(-rows // 8) * 8
    p = rows * _LANES
    b4 = jnp.pad(boxes.T, ((0, 0), (0, p - n))).reshape(4, rows, _LANES)
    sp = jnp.pad(scores, (0, p - n)).reshape(rows, _LANES)
    out = pl.pallas_call(
        functools.partial(_nms_kernel, rows=rows),
        out_shape=jax.ShapeDtypeStruct((_MAX_OUT, _LANES), jnp.float32),
        scratch_shapes=[pltpu.VMEM((rows, _LANES), jnp.float32)],
    )(b4, sp)
    return out[:, :5]


# speedup vs baseline: 44.6572x; 1.2096x over previous
import functools

import jax
import jax.numpy as jnp
from jax import lax
from jax.experimental import pallas as pl
from jax.experimental.pallas import tpu as pltpu

_SCORE_THR = 0.05
_MAX_OUT = 300
_LANES = 128


def _tree(op, sw, rows):
    cm = sw[0:8]
    for k in range(1, rows // 8):
        cm = op(cm, sw[8 * k:8 * k + 8])
    return cm


def _red2(red, m8):
    return red(red(m8, axis=0, keepdims=True), axis=1, keepdims=True)


def _nms_kernel(b_ref, s_ref, out_ref, sw_ref, area_ref, *, rows):
    x1 = b_ref[0]
    y1 = b_ref[1]
    x2 = b_ref[2]
    y2 = b_ref[3]
    area_ref[...] = (x2 - x1) * (y2 - y1)
    s = s_ref[...]
    sw0 = jnp.where(s > _SCORE_THR, s, 0.0)
    sw_ref[...] = sw0
    lin = (lax.broadcasted_iota(jnp.int32, (rows, _LANES), 0) * _LANES
           + lax.broadcasted_iota(jnp.int32, (rows, _LANES), 1))
    lane = lax.broadcasted_iota(jnp.int32, (1, _LANES), 1)

    def body(i, cm):
        vmax = _red2(jnp.max, cm)
        valid = vmax > _SCORE_THR
        sw = sw_ref[...]
        idx = _red2(jnp.min, _tree(jnp.minimum,
                                   jnp.where(sw == vmax, lin, jnp.int32(2**30)),
                                   rows))
        sel = lin == idx
        bx1 = _red2(jnp.sum, _tree(jnp.add, jnp.where(sel, x1, 0.0), rows))
        by1 = _red2(jnp.sum, _tree(jnp.add, jnp.where(sel, y1, 0.0), rows))
        bx2 = _red2(jnp.sum, _tree(jnp.add, jnp.where(sel, x2, 0.0), rows))
        by2 = _red2(jnp.sum, _tree(jnp.add, jnp.where(sel, y2, 0.0), rows))
        barea = (bx2 - bx1) * (by2 - by1)
        iw = jnp.maximum(jnp.minimum(bx2, x2) - jnp.maximum(bx1, x1), 0.0)
        ih = jnp.maximum(jnp.minimum(by2, y2) - jnp.maximum(by1, y1), 0.0)
        inter = iw * ih
        keep = (3.0 * inter < barea + area_ref[...] + 1e-6) | ~valid
        sw_new = jnp.where(keep, sw, 0.0)
        sw_ref[...] = sw_new
        det = (jnp.where(lane == 0, bx1, 0.0)
               + jnp.where(lane == 1, by1, 0.0)
               + jnp.where(lane == 2, bx2, 0.0)
               + jnp.where(lane == 3, by2, 0.0)
               + jnp.where(lane == 4, vmax, 0.0))
        out_ref[pl.ds(i, 1), :] = jnp.where(valid, det, 0.0)
        return _tree(jnp.maximum, sw_new, rows)

    lax.fori_loop(0, _MAX_OUT, body, _tree(jnp.maximum, sw0, rows))


def kernel(boxes, scores):
    n = boxes.shape[0]
    rows = -(-n // _LANES)
    rows = -(-rows // 8) * 8
    p = rows * _LANES
    b4 = jnp.pad(boxes.T, ((0, 0), (0, p - n))).reshape(4, rows, _LANES)
    sp = jnp.pad(scores, (0, p - n)).reshape(rows, _LANES)
    out = pl.pallas_call(
        functools.partial(_nms_kernel, rows=rows),
        out_shape=jax.ShapeDtypeStruct((_MAX_OUT, _LANES), jnp.float32),
        scratch_shapes=[pltpu.VMEM((rows, _LANES), jnp.float32),
                        pltpu.VMEM((rows, _LANES), jnp.float32)],
    )(b4, sp)
    return out[:, :5]


# speedup vs baseline: 50.8208x; 1.1380x over previous
import functools

import jax
import jax.numpy as jnp
from jax import lax
from jax.experimental import pallas as pl
from jax.experimental.pallas import tpu as pltpu

_SCORE_THR = 0.05
_MAX_OUT = 300
_LANES = 128
_K = 8


def _tree(op, sw, rows):
    cm = sw[0:8]
    for k in range(1, rows // 8):
        cm = op(cm, sw[8 * k:8 * k + 8])
    return cm


def _red2(red, m8):
    return red(red(m8, axis=0, keepdims=True), axis=1, keepdims=True)


def _nms_kernel(b_ref, s_ref, out_ref, sw_ref, area_ref, *, rows):
    x1 = b_ref[0]
    y1 = b_ref[1]
    x2 = b_ref[2]
    y2 = b_ref[3]
    area_ref[...] = (x2 - x1) * (y2 - y1)
    s = s_ref[...]
    sw_ref[...] = jnp.where(s > _SCORE_THR, s, 0.0)
    out_ref[...] = jnp.zeros_like(out_ref)
    lin = (lax.broadcasted_iota(jnp.int32, (rows, _LANES), 0) * _LANES
           + lax.broadcasted_iota(jnp.int32, (rows, _LANES), 1))
    lane = lax.broadcasted_iota(jnp.int32, (1, _LANES), 1)
    subi = lax.broadcasted_iota(jnp.int32, (8, 1), 0)

    def cond(c):
        cnt, alive = c
        return (cnt < _MAX_OUT) & (alive > 0)

    def body(c):
        cnt, alive = c
        sw = sw_ref[...]
        area = area_ref[...]

        swm = sw
        cand = []
        for _ in range(_K):
            g = _red2(jnp.max, _tree(jnp.maximum, swm, rows))
            eq = swm == g
            idx = _red2(jnp.min, _tree(jnp.minimum,
                                       jnp.where(eq, lin, jnp.int32(2**30)),
                                       rows))
            sel = lin == idx
            bx1 = _red2(jnp.sum, _tree(jnp.add, jnp.where(sel, x1, 0.0), rows))
            by1 = _red2(jnp.sum, _tree(jnp.add, jnp.where(sel, y1, 0.0), rows))
            bx2 = _red2(jnp.sum, _tree(jnp.add, jnp.where(sel, x2, 0.0), rows))
            by2 = _red2(jnp.sum, _tree(jnp.add, jnp.where(sel, y2, 0.0), rows))
            swm = jnp.where(sel, 0.0, swm)
            cand.append((g, bx1, by1, bx2, by2, g > _SCORE_THR))

        acc = []
        for j in range(_K):
            gj, jx1, jy1, jx2, jy2, vj = cand[j]
            aj = (jx2 - jx1) * (jy2 - jy1)
            a = vj
            for m in range(j):
                gm, mx1, my1, mx2, my2, _ = cand[m]
                am = (mx2 - mx1) * (my2 - my1)
                iw = jnp.maximum(jnp.minimum(mx2, jx2) - jnp.maximum(mx1, jx1),
                                 0.0)
                ih = jnp.maximum(jnp.minimum(my2, jy2) - jnp.maximum(my1, jy1),
                                 0.0)
                ovl = 3.0 * (iw * ih) >= am + aj + 1e-6
                a = a & ~(acc[m] & ovl)
            acc.append(a)

        keep = None
        for j in range(_K):
            gj, jx1, jy1, jx2, jy2, _ = cand[j]
            aj = (jx2 - jx1) * (jy2 - jy1)
            iw = jnp.maximum(jnp.minimum(jx2, x2) - jnp.maximum(jx1, x1), 0.0)
            ih = jnp.maximum(jnp.minimum(jy2, y2) - jnp.maximum(jy1, y1), 0.0)
            kill = acc[j] & (3.0 * (iw * ih) >= aj + area + 1e-6)
            keep = ~kill if keep is None else keep & ~kill
        sw_ref[...] = jnp.where(keep, sw, 0.0)

        blk = jnp.zeros((8, _LANES), jnp.float32)
        o = jnp.zeros((1, 1), jnp.int32)
        for j in range(_K):
            gj, jx1, jy1, jx2, jy2, _ = cand[j]
            det = (jnp.where(lane == 0, jx1, 0.0)
                   + jnp.where(lane == 1, jy1, 0.0)
                   + jnp.where(lane == 2, jx2, 0.0)
                   + jnp.where(lane == 3, jy2, 0.0)
                   + jnp.where(lane == 4, gj, 0.0))
            blk = blk + jnp.where((subi == o) & acc[j], det, 0.0)
            o = o + jnp.where(acc[j], 1, 0)
        out_ref[pl.ds(cnt, 8), :] = blk

        code = jnp.sum(o + jnp.where(cand[0][5], 16, 0))
        return cnt + (code & 15), code >> 4

    lax.while_loop(cond, body, (jnp.int32(0), jnp.int32(1)))


def kernel(boxes, scores):
    n = boxes.shape[0]
    rows = -(-n // _LANES)
    rows = -(-rows // 8) * 8
    p = rows * _LANES
    b4 = jnp.pad(boxes.T, ((0, 0), (0, p - n))).reshape(4, rows, _LANES)
    sp = jnp.pad(scores, (0, p - n)).reshape(rows, _LANES)
    out = pl.pallas_call(
        functools.partial(_nms_kernel, rows=rows),
        out_shape=jax.ShapeDtypeStruct((_MAX_OUT + 12, _LANES), jnp.float32),
        scratch_shapes=[pltpu.VMEM((rows, _LANES), jnp.float32),
                        pltpu.VMEM((rows, _LANES), jnp.float32)],
    )(b4, sp)
    return out[:_MAX_OUT, :5]
